# baseline (speedup 1.0000x reference)
import functools

import jax
import jax.numpy as jnp
from jax import lax
from jax.experimental import pallas as pl
from jax.experimental.pallas import tpu as pltpu
from jax.experimental.pallas import tpu_sc as plsc

VOCAB = 30522
HIDDEN = 768
MAX_POS = 512
EPS = 1e-12
B, S = 128, 512

L = 16
NJ = HIDDEN // L
NC, NS = 2, 16
NW = NC * NS
POS_PER_W = S // NW
CHUNK = 32
NCHUNK_B = B // CHUNK
NCH = POS_PER_W * NCHUNK_B
TB = 4


def _lane_sum(x):
    lanes = lax.iota(jnp.int32, L)
    dnums = lax.GatherDimensionNumbers(
        offset_dims=(), collapsed_slice_dims=(0,), start_index_map=(0,))
    for sh in (1, 2, 4, 8):
        idx = (lanes ^ sh).reshape(L, 1)
        x = x + lax.gather(x, idx, dnums, (1,),
                           mode=lax.GatherScatterMode.PROMISE_IN_BOUNDS)
    return x


def _rsqrt(x):
    i = plsc.bitcast(x, jnp.int32)
    i = jnp.int32(0x5F3759DF) - (i >> 1)
    y = plsc.bitcast(i, jnp.float32)
    half = x * jnp.float32(0.5)
    for _ in range(3):
        y = y * (jnp.float32(1.5) - half * y * y)
    return y


def _sc_body(ids_ref, word_ref, pos_ref, type_ref, gamma_ref, beta_ref,
             out_ref, idx0, idx1, g0, g1, o0, o1, bias_v, typerow_v,
             gamma_v, beta_v, gsem0, gsem1, osem0, osem1):
    wid = lax.axis_index("s") * NC + lax.axis_index("c")
    s_base = wid * POS_PER_W

    pltpu.sync_copy(gamma_ref, gamma_v)
    pltpu.sync_copy(beta_ref, beta_v)
    pltpu.sync_copy(type_ref.at[pl.ds(0, 1)], typerow_v)
    pltpu.sync_copy(pos_ref.at[pl.ds(s_base, POS_PER_W)], bias_v)

    def add_type(sl, _):
        for j in range(NJ):
            d = pl.ds(j * L, L)
            bias_v[sl, d] = bias_v[sl, d] + typerow_v[0, d]
        return _
    lax.fori_loop(0, POS_PER_W, add_type, None)

    inv_h = jnp.float32(1.0 / HIDDEN)

    def chunk_slices(c):
        s_local = c // NCHUNK_B
        b0 = (c % NCHUNK_B) * CHUNK
        return s_local, b0

    def stage_and_gather(c, idxbuf, gbuf, gsem):
        s_local, b0 = chunk_slices(c)
        off = (s_base + s_local) * B + b0
        pltpu.sync_copy(ids_ref.at[pl.ds(off, CHUNK)], idxbuf)
        pltpu.async_copy(word_ref.at[idxbuf], gbuf, gsem)

    def wait_gather(idxbuf, gbuf, gsem):
        pltpu.make_async_copy(word_ref.at[idxbuf], gbuf, gsem).wait()

    def start_scatter(c, obuf, osem):
        s_local, b0 = chunk_slices(c)
        s_col = (s_base + s_local) * HIDDEN
        pltpu.async_copy(obuf, out_ref.at[pl.ds(b0, CHUNK),
                                          pl.ds(s_col, HIDDEN)], osem)

    def wait_scatter(obuf, osem):
        pltpu.make_async_copy(obuf, out_ref.at[pl.ds(0, CHUNK),
                                               pl.ds(0, HIDDEN)],
                              osem).wait()

    def compute(c, gbuf, obuf):
        s_local, _ = chunk_slices(c)
        ABLATE_STATS = True

        def do_block(blk, _):
            t0 = blk * TB
            zero = jnp.zeros((L,), jnp.float32)
            s0 = [zero] * TB
            s1 = [zero] * TB
            for j in range(NJ):
                d = pl.ds(j * L, L)
                bj = bias_v[s_local, d]
                for t in range(TB):
                    x = gbuf[t0 + t, d] + bj
                    obuf[t0 + t, d] = x
                    s0[t] = s0[t] + x
                    s1[t] = s1[t] + x * x
            mean = []
            rstd = []
            for t in range(TB):
                if ABLATE_STATS:
                    mean.append(s0[t] * jnp.float32(0.0))
                    rstd.append(s1[t] * jnp.float32(0.0) + jnp.float32(1.0))
                    continue
                m = _lane_sum(s0[t]) * inv_h
                v = jnp.maximum(_lane_sum(s1[t]) * inv_h - m * m,
                                jnp.float32(0.0))
                mean.append(m)
                rstd.append(_rsqrt(v + jnp.float32(EPS)))
            for j in range(NJ):
                d = pl.ds(j * L, L)
                gj = gamma_v[d]
                bj = beta_v[d]
                for t in range(TB):
                    x = obuf[t0 + t, d]
                    obuf[t0 + t, d] = (x - mean[t]) * rstd[t] * gj + bj
            return _
        lax.fori_loop(0, CHUNK // TB, do_block, None)

    stage_and_gather(0, idx0, g0, gsem0)

    def pair(c2, _):
        c = c2 * 2
        stage_and_gather(c + 1, idx1, g1, gsem1)
        wait_gather(idx0, g0, gsem0)

        @pl.when(c2 > 0)
        def _wait_o0():
            wait_scatter(o0, osem0)
        compute(c, g0, o0)
        start_scatter(c, o0, osem0)

        @pl.when(c + 2 < NCH)
        def _next_g0():
            stage_and_gather(c + 2, idx0, g0, gsem0)
        wait_gather(idx1, g1, gsem1)

        @pl.when(c2 > 0)
        def _wait_o1():
            wait_scatter(o1, osem1)
        compute(c + 1, g1, o1)
        start_scatter(c + 1, o1, osem1)
        return _
    lax.fori_loop(0, NCH // 2, pair, None)
    wait_scatter(o0, osem0)
    wait_scatter(o1, osem1)


@functools.partial(jax.jit, static_argnames=())
def kernel(input_ids, attention_mask, labels, word_emb, pos_emb, type_emb,
           ln_gamma, ln_beta):
    del attention_mask
    ids_t = input_ids.T.reshape(-1)
    mesh = plsc.VectorSubcoreMesh(core_axis_name="c", subcore_axis_name="s")
    f = pl.kernel(
        _sc_body,
        out_type=jax.ShapeDtypeStruct((B, S * HIDDEN), jnp.float32),
        mesh=mesh,
        compiler_params=pltpu.CompilerParams(needs_layout_passes=False),
        scratch_types=[
            pltpu.VMEM((CHUNK,), jnp.int32),
            pltpu.VMEM((CHUNK,), jnp.int32),
            pltpu.VMEM((CHUNK, HIDDEN), jnp.float32),
            pltpu.VMEM((CHUNK, HIDDEN), jnp.float32),
            pltpu.VMEM((CHUNK, HIDDEN), jnp.float32),
            pltpu.VMEM((CHUNK, HIDDEN), jnp.float32),
            pltpu.VMEM((POS_PER_W, HIDDEN), jnp.float32),
            pltpu.VMEM((1, HIDDEN), jnp.float32),
            pltpu.VMEM((HIDDEN,), jnp.float32),
            pltpu.VMEM((HIDDEN,), jnp.float32),
            pltpu.SemaphoreType.DMA,
            pltpu.SemaphoreType.DMA,
            pltpu.SemaphoreType.DMA,
            pltpu.SemaphoreType.DMA,
        ],
    )
    out = f(ids_t, word_emb, pos_emb, type_emb, ln_gamma, ln_beta)
    return out.reshape(B, S, HIDDEN), labels


# speedup vs baseline: 2.1637x; 2.1637x over previous
import functools

import jax
import jax.numpy as jnp
from jax import lax
from jax.experimental import pallas as pl
from jax.experimental.pallas import tpu as pltpu
from jax.experimental.pallas import tpu_sc as plsc

VOCAB = 30522
HIDDEN = 768
MAX_POS = 512
EPS = 1e-12
B, S = 128, 512

L = 16
NJ = HIDDEN // L
NC, NS = 2, 16
NW = NC * NS
POS_PER_W = S // NW
CHUNK = 32
NCHUNK_B = B // CHUNK
NCH = POS_PER_W * NCHUNK_B
TB = 8


def _lane_sum(x):
    lanes = lax.iota(jnp.int32, L)
    dnums = lax.GatherDimensionNumbers(
        offset_dims=(), collapsed_slice_dims=(0,), start_index_map=(0,))
    for sh in (1, 2, 4, 8):
        idx = (lanes ^ sh).reshape(L, 1)
        x = x + lax.gather(x, idx, dnums, (1,),
                           mode=lax.GatherScatterMode.PROMISE_IN_BOUNDS)
    return x


def _rsqrt(x):
    i = plsc.bitcast(x, jnp.int32)
    i = jnp.int32(0x5F3759DF) - (i >> 1)
    y = plsc.bitcast(i, jnp.float32)
    half = x * jnp.float32(0.5)
    for _ in range(3):
        y = y * (jnp.float32(1.5) - half * y * y)
    return y


def _sc_body(ids_ref, word_ref, pos_ref, type_ref, gamma_ref, beta_ref,
             out_ref, idx0, idx1, g0, g1, o0, o1, bias_v, typerow_v,
             gamma_v, beta_v, gsem0, gsem1, osem0, osem1):
    wid = lax.axis_index("s") * NC + lax.axis_index("c")
    s_base = wid * POS_PER_W

    pltpu.sync_copy(gamma_ref, gamma_v)
    pltpu.sync_copy(beta_ref, beta_v)
    pltpu.sync_copy(type_ref.at[pl.ds(0, 1)], typerow_v)
    pltpu.sync_copy(pos_ref.at[pl.ds(s_base, POS_PER_W)], bias_v)

    def add_type(sl, _):
        for j in range(NJ):
            d = pl.ds(j * L, L)
            bias_v[sl, d] = bias_v[sl, d] + typerow_v[0, d]
        return _
    lax.fori_loop(0, POS_PER_W, add_type, None)

    inv_h = jnp.float32(1.0 / HIDDEN)

    def chunk_slices(c):
        s_local = c // NCHUNK_B
        b0 = (c % NCHUNK_B) * CHUNK
        return s_local, b0

    def stage_and_gather(c, idxbuf, gbuf, gsem):
        s_local, b0 = chunk_slices(c)
        off = (s_base + s_local) * B + b0
        pltpu.sync_copy(ids_ref.at[pl.ds(off, CHUNK)], idxbuf)
        pltpu.async_copy(word_ref.at[idxbuf], gbuf, gsem)

    def wait_gather(idxbuf, gbuf, gsem):
        pltpu.make_async_copy(word_ref.at[idxbuf], gbuf, gsem).wait()

    def start_scatter(c, obuf, osem):
        s_local, b0 = chunk_slices(c)
        s_col = (s_base + s_local) * HIDDEN
        pltpu.async_copy(obuf, out_ref.at[pl.ds(b0, CHUNK),
                                          pl.ds(s_col, HIDDEN)], osem)

    def wait_scatter(obuf, osem):
        pltpu.make_async_copy(obuf, out_ref.at[pl.ds(0, CHUNK),
                                               pl.ds(0, HIDDEN)],
                              osem).wait()

    def compute(c, gbuf, obuf):
        s_local, _ = chunk_slices(c)
        ABLATE_STATS = False

        def do_block(blk, _):
            t0 = blk * TB
            zero = jnp.zeros((L,), jnp.float32)
            s0 = [zero] * TB
            s1 = [zero] * TB
            for j in range(NJ):
                d = pl.ds(j * L, L)
                bj = bias_v[s_local, d]
                for t in range(TB):
                    x = gbuf[t0 + t, d] + bj
                    obuf[t0 + t, d] = x
                    s0[t] = s0[t] + x
                    s1[t] = s1[t] + x * x
            mean = []
            rstd = []
            for t in range(TB):
                if ABLATE_STATS:
                    mean.append(s0[t] * jnp.float32(0.0))
                    rstd.append(s1[t] * jnp.float32(0.0) + jnp.float32(1.0))
                    continue
                m = _lane_sum(s0[t]) * inv_h
                v = jnp.maximum(_lane_sum(s1[t]) * inv_h - m * m,
                                jnp.float32(0.0))
                mean.append(m)
                rstd.append(_rsqrt(v + jnp.float32(EPS)))
            for j in range(NJ):
                d = pl.ds(j * L, L)
                gj = gamma_v[d]
                bj = beta_v[d]
                for t in range(TB):
                    x = obuf[t0 + t, d]
                    obuf[t0 + t, d] = (x - mean[t]) * rstd[t] * gj + bj
            return _
        lax.fori_loop(0, CHUNK // TB, do_block, None)

    stage_and_gather(0, idx0, g0, gsem0)

    def pair(c2, _):
        c = c2 * 2
        stage_and_gather(c + 1, idx1, g1, gsem1)
        wait_gather(idx0, g0, gsem0)

        @pl.when(c2 > 0)
        def _wait_o0():
            wait_scatter(o0, osem0)
        compute(c, g0, o0)
        start_scatter(c, o0, osem0)

        @pl.when(c + 2 < NCH)
        def _next_g0():
            stage_and_gather(c + 2, idx0, g0, gsem0)
        wait_gather(idx1, g1, gsem1)

        @pl.when(c2 > 0)
        def _wait_o1():
            wait_scatter(o1, osem1)
        compute(c + 1, g1, o1)
        start_scatter(c + 1, o1, osem1)
        return _
    lax.fori_loop(0, NCH // 2, pair, None)
    wait_scatter(o0, osem0)
    wait_scatter(o1, osem1)


@functools.partial(jax.jit, static_argnames=())
def kernel(input_ids, attention_mask, labels, word_emb, pos_emb, type_emb,
           ln_gamma, ln_beta):
    del attention_mask
    ids_t = input_ids.T.reshape(-1)
    mesh = plsc.VectorSubcoreMesh(core_axis_name="c", subcore_axis_name="s")
    f = pl.kernel(
        _sc_body,
        out_type=jax.ShapeDtypeStruct((B, S * HIDDEN), jnp.float32),
        mesh=mesh,
        compiler_params=pltpu.CompilerParams(needs_layout_passes=False,
                                             disable_bounds_checks=True),
        scratch_types=[
            pltpu.VMEM((CHUNK,), jnp.int32),
            pltpu.VMEM((CHUNK,), jnp.int32),
            pltpu.VMEM((CHUNK, HIDDEN), jnp.float32),
            pltpu.VMEM((CHUNK, HIDDEN), jnp.float32),
            pltpu.VMEM((CHUNK, HIDDEN), jnp.float32),
            pltpu.VMEM((CHUNK, HIDDEN), jnp.float32),
            pltpu.VMEM((POS_PER_W, HIDDEN), jnp.float32),
            pltpu.VMEM((1, HIDDEN), jnp.float32),
            pltpu.VMEM((HIDDEN,), jnp.float32),
            pltpu.VMEM((HIDDEN,), jnp.float32),
            pltpu.SemaphoreType.DMA,
            pltpu.SemaphoreType.DMA,
            pltpu.SemaphoreType.DMA,
            pltpu.SemaphoreType.DMA,
        ],
    )
    out = f(ids_t, word_emb, pos_emb, type_emb, ln_gamma, ln_beta)
    return out.reshape(B, S, HIDDEN), labels


# speedup vs baseline: 2.1742x; 1.0049x over previous
import functools

import jax
import jax.numpy as jnp
from jax import lax
from jax.experimental import pallas as pl
from jax.experimental.pallas import tpu as pltpu
from jax.experimental.pallas import tpu_sc as plsc

VOCAB = 30522
HIDDEN = 768
MAX_POS = 512
EPS = 1e-12
B, S = 128, 512

L = 16
NJ = HIDDEN // L
NC, NS = 2, 16
NW = NC * NS
POS_PER_W = S // NW
CHUNK = 32
NCHUNK_B = B // CHUNK
NCH = POS_PER_W * NCHUNK_B
TB = 8


def _lane_sum(x):
    lanes = lax.iota(jnp.int32, L)
    dnums = lax.GatherDimensionNumbers(
        offset_dims=(), collapsed_slice_dims=(0,), start_index_map=(0,))
    for sh in (1, 2, 4, 8):
        idx = (lanes ^ sh).reshape(L, 1)
        x = x + lax.gather(x, idx, dnums, (1,),
                           mode=lax.GatherScatterMode.PROMISE_IN_BOUNDS)
    return x


def _rsqrt(x):
    i = plsc.bitcast(x, jnp.int32)
    i = jnp.int32(0x5F3759DF) - (i >> 1)
    y = plsc.bitcast(i, jnp.float32)
    half = x * jnp.float32(0.5)
    for _ in range(3):
        y = y * (jnp.float32(1.5) - half * y * y)
    return y


def _sc_body(ids_ref, word_ref, pos_ref, type_ref, gamma_ref, beta_ref,
             out_ref, idx0, idx1, g0, g1, o0, o1, bias_v, typerow_v,
             gamma_v, beta_v, gsem0, gsem1, osem0, osem1):
    wid = lax.axis_index("s") * NC + lax.axis_index("c")
    s_base = wid * POS_PER_W

    pltpu.sync_copy(gamma_ref, gamma_v)
    pltpu.sync_copy(beta_ref, beta_v)
    pltpu.sync_copy(type_ref.at[pl.ds(0, 1)], typerow_v)
    pltpu.sync_copy(pos_ref.at[pl.ds(s_base, POS_PER_W)], bias_v)

    @plsc.parallel_loop(0, POS_PER_W)
    def add_type(sl):
        for j in range(NJ):
            d = pl.ds(j * L, L)
            bias_v[sl, d] = bias_v[sl, d] + typerow_v[0, d]

    inv_h = jnp.float32(1.0 / HIDDEN)

    def chunk_slices(c):
        s_local = c // NCHUNK_B
        b0 = (c % NCHUNK_B) * CHUNK
        return s_local, b0

    def stage_and_gather(c, idxbuf, gbuf, gsem):
        s_local, b0 = chunk_slices(c)
        off = (s_base + s_local) * B + b0
        pltpu.sync_copy(ids_ref.at[pl.ds(off, CHUNK)], idxbuf)
        pltpu.async_copy(word_ref.at[idxbuf], gbuf, gsem)

    def wait_gather(idxbuf, gbuf, gsem):
        pltpu.make_async_copy(word_ref.at[idxbuf], gbuf, gsem).wait()

    def start_scatter(c, obuf, osem):
        s_local, b0 = chunk_slices(c)
        s_col = (s_base + s_local) * HIDDEN
        pltpu.async_copy(obuf, out_ref.at[pl.ds(b0, CHUNK),
                                          pl.ds(s_col, HIDDEN)], osem)

    def wait_scatter(obuf, osem):
        pltpu.make_async_copy(obuf, out_ref.at[pl.ds(0, CHUNK),
                                               pl.ds(0, HIDDEN)],
                              osem).wait()

    def compute(c, gbuf, obuf):
        s_local, _ = chunk_slices(c)
        ABLATE_STATS = False

        @plsc.parallel_loop(0, CHUNK // TB)
        def do_block(blk):
            t0 = blk * TB
            zero = jnp.zeros((L,), jnp.float32)
            s0 = [zero] * TB
            s1 = [zero] * TB
            for j in range(NJ):
                d = pl.ds(j * L, L)
                bj = bias_v[s_local, d]
                for t in range(TB):
                    x = gbuf[t0 + t, d] + bj
                    obuf[t0 + t, d] = x
                    s0[t] = s0[t] + x
                    s1[t] = s1[t] + x * x
            mean = []
            rstd = []
            for t in range(TB):
                if ABLATE_STATS:
                    mean.append(s0[t] * jnp.float32(0.0))
                    rstd.append(s1[t] * jnp.float32(0.0) + jnp.float32(1.0))
                    continue
                m = _lane_sum(s0[t]) * inv_h
                v = jnp.maximum(_lane_sum(s1[t]) * inv_h - m * m,
                                jnp.float32(0.0))
                mean.append(m)
                rstd.append(_rsqrt(v + jnp.float32(EPS)))
            for j in range(NJ):
                d = pl.ds(j * L, L)
                gj = gamma_v[d]
                bj = beta_v[d]
                for t in range(TB):
                    x = obuf[t0 + t, d]
                    obuf[t0 + t, d] = (x - mean[t]) * rstd[t] * gj + bj

    stage_and_gather(0, idx0, g0, gsem0)

    def pair(c2, _):
        c = c2 * 2
        stage_and_gather(c + 1, idx1, g1, gsem1)
        wait_gather(idx0, g0, gsem0)

        @pl.when(c2 > 0)
        def _wait_o0():
            wait_scatter(o0, osem0)
        compute(c, g0, o0)
        start_scatter(c, o0, osem0)

        @pl.when(c + 2 < NCH)
        def _next_g0():
            stage_and_gather(c + 2, idx0, g0, gsem0)
        wait_gather(idx1, g1, gsem1)

        @pl.when(c2 > 0)
        def _wait_o1():
            wait_scatter(o1, osem1)
        compute(c + 1, g1, o1)
        start_scatter(c + 1, o1, osem1)
        return _
    lax.fori_loop(0, NCH // 2, pair, None)
    wait_scatter(o0, osem0)
    wait_scatter(o1, osem1)


@functools.partial(jax.jit, static_argnames=())
def kernel(input_ids, attention_mask, labels, word_emb, pos_emb, type_emb,
           ln_gamma, ln_beta):
    del attention_mask
    ids_t = input_ids.T.reshape(-1)
    mesh = plsc.VectorSubcoreMesh(core_axis_name="c", subcore_axis_name="s")
    f = pl.kernel(
        _sc_body,
        out_type=jax.ShapeDtypeStruct((B, S * HIDDEN), jnp.float32),
        mesh=mesh,
        compiler_params=pltpu.CompilerParams(needs_layout_passes=False,
                                             disable_bounds_checks=True),
        scratch_types=[
            pltpu.VMEM((CHUNK,), jnp.int32),
            pltpu.VMEM((CHUNK,), jnp.int32),
            pltpu.VMEM((CHUNK, HIDDEN), jnp.float32),
            pltpu.VMEM((CHUNK, HIDDEN), jnp.float32),
            pltpu.VMEM((CHUNK, HIDDEN), jnp.float32),
            pltpu.VMEM((CHUNK, HIDDEN), jnp.float32),
            pltpu.VMEM((POS_PER_W, HIDDEN), jnp.float32),
            pltpu.VMEM((1, HIDDEN), jnp.float32),
            pltpu.VMEM((HIDDEN,), jnp.float32),
            pltpu.VMEM((HIDDEN,), jnp.float32),
            pltpu.SemaphoreType.DMA,
            pltpu.SemaphoreType.DMA,
            pltpu.SemaphoreType.DMA,
            pltpu.SemaphoreType.DMA,
        ],
    )
    out = f(ids_t, word_emb, pos_emb, type_emb, ln_gamma, ln_beta)
    return out.reshape(B, S, HIDDEN), labels


# speedup vs baseline: 6.6481x; 3.0577x over previous
import functools

import jax
import jax.numpy as jnp
from jax import lax
from jax.experimental import pallas as pl
from jax.experimental.pallas import tpu as pltpu
from jax.experimental.pallas import tpu_sc as plsc

VOCAB = 30522
HIDDEN = 768
MAX_POS = 512
EPS = 1e-12
B, S = 128, 512

NC, NS = 2, 16
NW = NC * NS
P = 4
BP = B // P
TOK_P = BP * S
TOK_W = TOK_P // NW
GCH = 32
NGC = TOK_W // GCH
NBUF = 4

BS_S = 8
TC_GRID = S // BS_S


def _sc_gather_body(ids_ref, word_ref, tmp_ref, idx_all, bufs, gsems, wsems):
    wid = lax.axis_index("s") * NC + lax.axis_index("c")
    base = wid * TOK_W

    pltpu.sync_copy(ids_ref.at[pl.ds(wid * NGC, NGC)], idx_all)

    def gather(c):
        pltpu.async_copy(word_ref.at[idx_all.at[c]], bufs.at[c % NBUF],
                         gsems.at[c % NBUF])

    def wait_gather(c):
        pltpu.make_async_copy(word_ref.at[idx_all.at[c]], bufs.at[c % NBUF],
                              gsems.at[c % NBUF]).wait()

    def write(c):
        pltpu.async_copy(bufs.at[c % NBUF],
                         tmp_ref.at[pl.ds(base + c * GCH, GCH)],
                         wsems.at[c % NBUF])

    def wait_write(c):
        pltpu.make_async_copy(bufs.at[c % NBUF],
                              tmp_ref.at[pl.ds(base + c * GCH, GCH)],
                              wsems.at[c % NBUF]).wait()

    for c in range(NGC):
        if c >= NBUF - 1:
            wait_write(c - (NBUF - 1))
        gather(c)
        if c >= 1:
            wait_gather(c - 1)
            write(c - 1)
    wait_gather(NGC - 1)
    write(NGC - 1)
    for c in range(NGC - (NBUF - 1), NGC):
        wait_write(c)


def _make_sc_gather():
    mesh = plsc.VectorSubcoreMesh(core_axis_name="c", subcore_axis_name="s")
    return pl.kernel(
        _sc_gather_body,
        out_type=jax.ShapeDtypeStruct((TOK_P, HIDDEN), jnp.float32),
        mesh=mesh,
        compiler_params=pltpu.CompilerParams(needs_layout_passes=False),
        scratch_types=[
            pltpu.VMEM((NGC, GCH), jnp.int32),
            pltpu.VMEM((NBUF, GCH, HIDDEN), jnp.float32),
            pltpu.SemaphoreType.DMA((NBUF,)),
            pltpu.SemaphoreType.DMA((NBUF,)),
        ],
    )


def _tc_ln_body(tmp_ref, pos_ref, type_ref, gamma_ref, beta_ref, out_ref):
    x = tmp_ref[...]
    bias = pos_ref[...] + type_ref[...]
    y = x + bias[None, :, :]
    mean = jnp.mean(y, axis=-1, keepdims=True)
    var = jnp.mean(jnp.square(y - mean), axis=-1, keepdims=True)
    normed = (y - mean) * lax.rsqrt(var + jnp.float32(EPS))
    out_ref[...] = normed * gamma_ref[...][None, :, :] + beta_ref[...][None]


def _tc_ln_alias_body(out_in_ref, tmp_ref, pos_ref, type_ref, gamma_ref,
                      beta_ref, out_ref):
    del out_in_ref
    _tc_ln_body(tmp_ref, pos_ref, type_ref, gamma_ref, beta_ref, out_ref)


def _tc_specs(p):
    in_specs = [
        pl.BlockSpec((BP, BS_S, HIDDEN), lambda i: (0, i, 0)),
        pl.BlockSpec((BS_S, HIDDEN), lambda i: (i, 0)),
        pl.BlockSpec((1, HIDDEN), lambda i: (0, 0)),
        pl.BlockSpec((1, HIDDEN), lambda i: (0, 0)),
        pl.BlockSpec((1, HIDDEN), lambda i: (0, 0)),
    ]
    out_spec = pl.BlockSpec((BP, BS_S, HIDDEN), lambda i, p=p: (p, i, 0))
    return in_specs, out_spec


def _make_tc_ln(p, aliased):
    in_specs, out_spec = _tc_specs(p)
    if aliased:
        in_specs = [pl.BlockSpec(memory_space=pl.ANY)] + in_specs
    return pl.pallas_call(
        _tc_ln_alias_body if aliased else _tc_ln_body,
        grid=(TC_GRID,),
        in_specs=in_specs,
        out_specs=out_spec,
        out_shape=jax.ShapeDtypeStruct((B, S, HIDDEN), jnp.float32),
        input_output_aliases={0: 0} if aliased else {},
    )


@functools.partial(jax.jit, static_argnames=())
def kernel(input_ids, attention_mask, labels, word_emb, pos_emb, type_emb,
           ln_gamma, ln_beta):
    del attention_mask
    ids_rows = input_ids.reshape(-1, GCH)
    pos2 = pos_emb[:S]
    type2 = type_emb[0:1]
    gamma2 = ln_gamma.reshape(1, HIDDEN)
    beta2 = ln_beta.reshape(1, HIDDEN)
    sc_gather = _make_sc_gather()
    rows_per_phase = TOK_P // GCH
    out = None
    for p in range(P):
        ids_p = lax.slice_in_dim(ids_rows, p * rows_per_phase,
                                 (p + 1) * rows_per_phase, axis=0)
        tmp_p = sc_gather(ids_p, word_emb)
        tmp_p = tmp_p.reshape(BP, S, HIDDEN)
        if out is None:
            out = _make_tc_ln(p, False)(tmp_p, pos2, type2, gamma2, beta2)
        else:
            out = _make_tc_ln(p, True)(out, tmp_p, pos2, type2, gamma2,
                                       beta2)
    return out, labels
